# baseline (speedup 1.0000x reference)
import functools
import math

import jax
import jax.numpy as jnp
import numpy as np
from jax import lax
from jax.experimental import pallas as pl
from jax.experimental.pallas import tpu as pltpu
from jax.experimental.pallas import tpu_sc as plsc

_SCALE = 0.125
_NUM_BUCKETS = 32
_MAX_DISTANCE = 128
_HEADS = 12
_SEQ = 2048
_BLK = 256
_NBLK = _SEQ // _BLK
_NTYPES = 2
_NBAND = 2 * _NBLK - 1

_SC_B = _NTYPES * _BLK * _BLK
_SC_WORKERS = 32
_SC_PER_W = _SC_B // _SC_WORKERS
_SC_L = 16


def _bucket_lut() -> np.ndarray:
    bi = np.arange(_BLK, dtype=np.int64)[:, None]
    bj = np.arange(_BLK, dtype=np.int64)[None, :]
    max_exact = _NUM_BUCKETS // 2
    luts = []
    for off in (0, _BLK):
        n = np.maximum(bi - bj + off, 0)
        nf = np.maximum(n, 1).astype(np.float32)
        val = max_exact + (
            np.log(nf / np.float32(max_exact))
            / np.float32(math.log(_MAX_DISTANCE / max_exact))
            * np.float32(_NUM_BUCKETS - max_exact)
        ).astype(np.int32)
        val = np.minimum(val, _NUM_BUCKETS - 1)
        luts.append(np.where(n < max_exact, n, val).astype(np.int32))
    return np.stack(luts)


_LUT = _bucket_lut().reshape(_SC_B)


def _sc_gather_bias(tableT_flat, lut_flat):
    mesh = plsc.VectorSubcoreMesh(core_axis_name="c", subcore_axis_name="s")

    @functools.partial(
        pl.kernel,
        mesh=mesh,
        compiler_params=pltpu.CompilerParams(
            use_tc_tiling_on_sc=False, needs_layout_passes=False
        ),
        out_type=jax.ShapeDtypeStruct((_HEADS, _SC_B), jnp.float32),
        scratch_types=[
            pltpu.VMEM((_NUM_BUCKETS * _HEADS,), jnp.float32),
            pltpu.VMEM((_SC_PER_W,), jnp.int32),
            pltpu.VMEM((_HEADS, _SC_PER_W), jnp.float32),
            pltpu.SemaphoreType.DMA,
        ],
    )
    def body(tab_hbm, lut_hbm, out_hbm, tab_v, lut_v, stage_v, sem):
        wid = lax.axis_index("s") * 2 + lax.axis_index("c")
        base = pl.multiple_of(wid * _SC_PER_W, _SC_PER_W)
        pltpu.sync_copy(tab_hbm, tab_v)
        pltpu.sync_copy(lut_hbm.at[pl.ds(base, _SC_PER_W)], lut_v)

        def one(k, carry):
            o = pl.multiple_of(k * _SC_L, _SC_L)
            idx = lut_v[pl.ds(o, _SC_L)]
            for h in range(_HEADS):
                stage_v[h, pl.ds(o, _SC_L)] = plsc.load_gather(
                    tab_v, [idx + h * _NUM_BUCKETS]
                )
            return carry

        lax.fori_loop(0, _SC_PER_W // _SC_L, one, 0)
        copies = [
            pltpu.async_copy(
                stage_v.at[h], out_hbm.at[h, pl.ds(base, _SC_PER_W)], sem
            )
            for h in range(_HEADS)
        ]
        for cp in copies:
            cp.wait()

    return body(tableT_flat, lut_flat)


_ROWS = 1024
_RSUB = _ROWS // _BLK


def _tc_const_body(qk_ref, c_ref, out_ref):
    ib = pl.program_id(1)
    c0 = c_ref[0, 0, 0]
    c31 = c_ref[0, 0, 1]
    for si in range(_RSUB):
        i = ib * _RSUB + si
        rs = pl.ds(si * _BLK, _BLK)
        for j in range(_NBLK):
            d = i - j
            in_band = jnp.logical_or(d == 0, d == 1)
            add = jnp.where(in_band, 0.0, jnp.where(d < 0, c0, c31))
            sl = pl.ds(j * _BLK, _BLK)
            out_ref[0, 0, rs, sl] = qk_ref[0, 0, rs, sl] + add


def _tc_const(qk, consts):
    return pl.pallas_call(
        _tc_const_body,
        grid=(_HEADS, _SEQ // _ROWS),
        in_specs=[
            pl.BlockSpec((1, 1, _ROWS, _SEQ), lambda h, i: (0, h, i, 0)),
            pl.BlockSpec(
                (1, 1, 2), lambda h, i: (h, 0, 0), memory_space=pltpu.SMEM
            ),
        ],
        out_specs=pl.BlockSpec((1, 1, _ROWS, _SEQ), lambda h, i: (0, h, i, 0)),
        out_shape=jax.ShapeDtypeStruct(qk.shape, qk.dtype),
    )(qk, consts)


def _tc_band_body(acc_ref, bias_ref, out_ref):
    k = pl.program_id(1)
    t = jnp.where(k < _NBLK, 0, 1)
    out_ref[...] = acc_ref[...] + bias_ref[0, t]


def _tc_band(acc, bias_blocks):
    def _ij(h, k):
        i = jnp.where(k < _NBLK, k, k - _NBLK + 1)
        j = jnp.where(k < _NBLK, k, k - _NBLK)
        return (0, h, i, j)

    spec = pl.BlockSpec((1, 1, _BLK, _BLK), _ij)
    return pl.pallas_call(
        _tc_band_body,
        grid=(_HEADS, _NBAND),
        in_specs=[
            spec,
            pl.BlockSpec((1, _NTYPES, _BLK, _BLK), lambda h, k: (h, 0, 0, 0)),
        ],
        out_specs=spec,
        out_shape=jax.ShapeDtypeStruct(acc.shape, acc.dtype),
        input_output_aliases={0: 0},
    )(acc, bias_blocks)


def kernel(qk_dots, table):
    tableT = jnp.reshape(jnp.transpose(table * _SCALE), (_HEADS * _NUM_BUCKETS,))
    rows = _sc_gather_bias(tableT, jnp.asarray(_LUT))
    bias = rows.reshape(_HEADS, _NTYPES, _BLK, _BLK)
    consts = _SCALE * jnp.stack(
        [table[0], table[_NUM_BUCKETS - 1]], axis=1
    ).reshape(_HEADS, 1, 2)
    acc = _tc_const(qk_dots, consts)
    return _tc_band(acc, bias)


# speedup vs baseline: 1.0008x; 1.0008x over previous
import functools
import math

import jax
import jax.numpy as jnp
import numpy as np
from jax import lax
from jax.experimental import pallas as pl
from jax.experimental.pallas import tpu as pltpu
from jax.experimental.pallas import tpu_sc as plsc

_SCALE = 0.125
_NUM_BUCKETS = 32
_MAX_DISTANCE = 128
_HEADS = 12
_SEQ = 2048
_BLK = 256
_NBLK = _SEQ // _BLK
_NTYPES = 2
_NBAND = 2 * _NBLK - 1

_SC_B = _NTYPES * _BLK * _BLK
_SC_WORKERS = 32
_SC_PER_W = _SC_B // _SC_WORKERS
_SC_L = 16


def _bucket_lut() -> np.ndarray:
    bi = np.arange(_BLK, dtype=np.int64)[:, None]
    bj = np.arange(_BLK, dtype=np.int64)[None, :]
    max_exact = _NUM_BUCKETS // 2
    luts = []
    for off in (0, _BLK):
        n = np.maximum(bi - bj + off, 0)
        nf = np.maximum(n, 1).astype(np.float32)
        val = max_exact + (
            np.log(nf / np.float32(max_exact))
            / np.float32(math.log(_MAX_DISTANCE / max_exact))
            * np.float32(_NUM_BUCKETS - max_exact)
        ).astype(np.int32)
        val = np.minimum(val, _NUM_BUCKETS - 1)
        luts.append(np.where(n < max_exact, n, val).astype(np.int32))
    return np.stack(luts)


_LUT = _bucket_lut().reshape(_SC_B)


def _sc_gather_bias(tableT_flat, lut_flat):
    mesh = plsc.VectorSubcoreMesh(core_axis_name="c", subcore_axis_name="s")

    @functools.partial(
        pl.kernel,
        mesh=mesh,
        compiler_params=pltpu.CompilerParams(
            use_tc_tiling_on_sc=False, needs_layout_passes=False
        ),
        out_type=jax.ShapeDtypeStruct((_HEADS, _SC_B), jnp.float32),
        scratch_types=[
            pltpu.VMEM((_NUM_BUCKETS * _HEADS,), jnp.float32),
            pltpu.VMEM((_SC_PER_W,), jnp.int32),
            pltpu.VMEM((_HEADS, _SC_PER_W), jnp.float32),
            pltpu.SemaphoreType.DMA,
        ],
    )
    def body(tab_hbm, lut_hbm, out_hbm, tab_v, lut_v, stage_v, sem):
        wid = lax.axis_index("s") * 2 + lax.axis_index("c")
        base = pl.multiple_of(wid * _SC_PER_W, _SC_PER_W)
        pltpu.sync_copy(tab_hbm, tab_v)
        pltpu.sync_copy(lut_hbm.at[pl.ds(base, _SC_PER_W)], lut_v)

        def one(k, carry):
            o = pl.multiple_of(k * _SC_L, _SC_L)
            idx = lut_v[pl.ds(o, _SC_L)]
            for h in range(_HEADS):
                stage_v[h, pl.ds(o, _SC_L)] = plsc.load_gather(
                    tab_v, [idx + h * _NUM_BUCKETS]
                )
            return carry

        lax.fori_loop(0, _SC_PER_W // _SC_L, one, 0)
        copies = [
            pltpu.async_copy(
                stage_v.at[h], out_hbm.at[h, pl.ds(base, _SC_PER_W)], sem
            )
            for h in range(_HEADS)
        ]
        for cp in copies:
            cp.wait()

    return body(tableT_flat, lut_flat)


_ROWS = 1024
_RSUB = _ROWS // _BLK


def _tc_const_body(qk_ref, c_ref, out_ref):
    ib = pl.program_id(1)
    c0 = c_ref[0, 0, 0]
    c31 = c_ref[0, 0, 1]
    for si in range(_RSUB):
        i = ib * _RSUB + si
        rs = pl.ds(si * _BLK, _BLK)
        for j in range(_NBLK):
            d = i - j
            in_band = jnp.logical_or(d == 0, d == 1)
            add = jnp.where(in_band, 0.0, jnp.where(d < 0, c0, c31))
            sl = pl.ds(j * _BLK, _BLK)
            out_ref[0, 0, rs, sl] = qk_ref[0, 0, rs, sl] + add


def _tc_const(qk, consts):
    return pl.pallas_call(
        _tc_const_body,
        grid=(_HEADS, _SEQ // _ROWS),
        in_specs=[
            pl.BlockSpec((1, 1, _ROWS, _SEQ), lambda h, i: (0, h, i, 0)),
            pl.BlockSpec(
                (1, 1, 2), lambda h, i: (h, 0, 0), memory_space=pltpu.SMEM
            ),
        ],
        out_specs=pl.BlockSpec((1, 1, _ROWS, _SEQ), lambda h, i: (0, h, i, 0)),
        out_shape=jax.ShapeDtypeStruct(qk.shape, qk.dtype),
    )(qk, consts)


def _tc_band_body(acc_hbm_ref, qk_ref, bias_ref, out_ref):
    del acc_hbm_ref
    k = pl.program_id(1)
    t = jnp.where(k < _NBLK, 0, 1)
    out_ref[...] = qk_ref[...] + bias_ref[0, t]


def _tc_band(acc, qk, bias_blocks):
    def _ij(h, k):
        i = jnp.where(k < _NBLK, k, k - _NBLK + 1)
        j = jnp.where(k < _NBLK, k, k - _NBLK)
        return (0, h, i, j)

    spec = pl.BlockSpec((1, 1, _BLK, _BLK), _ij)
    return pl.pallas_call(
        _tc_band_body,
        grid=(_HEADS, _NBAND),
        in_specs=[
            pl.BlockSpec(memory_space=pl.ANY),
            spec,
            pl.BlockSpec((1, _NTYPES, _BLK, _BLK), lambda h, k: (h, 0, 0, 0)),
        ],
        out_specs=spec,
        out_shape=jax.ShapeDtypeStruct(acc.shape, acc.dtype),
        input_output_aliases={0: 0},
    )(acc, qk, bias_blocks)


def kernel(qk_dots, table):
    tableT = jnp.reshape(jnp.transpose(table * _SCALE), (_HEADS * _NUM_BUCKETS,))
    rows = _sc_gather_bias(tableT, jnp.asarray(_LUT))
    bias = rows.reshape(_HEADS, _NTYPES, _BLK, _BLK)
    consts = _SCALE * jnp.stack(
        [table[0], table[_NUM_BUCKETS - 1]], axis=1
    ).reshape(_HEADS, 1, 2)
    acc = _tc_const(qk_dots, consts)
    return _tc_band(acc, qk_dots, bias)


# speedup vs baseline: 1.3392x; 1.3381x over previous
import functools
import math

import jax
import jax.numpy as jnp
import numpy as np
from jax import lax
from jax.experimental import pallas as pl
from jax.experimental.pallas import tpu as pltpu
from jax.experimental.pallas import tpu_sc as plsc

_SCALE = 0.125
_NUM_BUCKETS = 32
_MAX_DISTANCE = 128
_HEADS = 12
_SEQ = 2048
_BLK = 256
_NBLK = _SEQ // _BLK
_NTYPES = 4
_HPAD = 16

_SC_B = _NTYPES * _BLK * _BLK
_SC_WORKERS = 32
_SC_PER_W = _SC_B // _SC_WORKERS
_SC_L = 16


def _bucket_lut() -> np.ndarray:
    bi = np.arange(_BLK, dtype=np.int64)[:, None]
    bj = np.arange(_BLK, dtype=np.int64)[None, :]
    max_exact = _NUM_BUCKETS // 2
    luts = []
    for off in (0, _BLK):
        n = np.maximum(bi - bj + off, 0)
        nf = np.maximum(n, 1).astype(np.float32)
        val = max_exact + (
            np.log(nf / np.float32(max_exact))
            / np.float32(math.log(_MAX_DISTANCE / max_exact))
            * np.float32(_NUM_BUCKETS - max_exact)
        ).astype(np.int32)
        val = np.minimum(val, _NUM_BUCKETS - 1)
        luts.append(np.where(n < max_exact, n, val).astype(np.int32))
    luts.append(np.zeros((_BLK, _BLK), np.int32))
    luts.append(np.full((_BLK, _BLK), _NUM_BUCKETS - 1, np.int32))
    return np.stack(luts)


def _sc_gather_bias(tableT_flat, lut_flat):
    mesh = plsc.VectorSubcoreMesh(core_axis_name="c", subcore_axis_name="s")

    @functools.partial(
        pl.kernel,
        mesh=mesh,
        compiler_params=pltpu.CompilerParams(
            use_tc_tiling_on_sc=False, needs_layout_passes=False
        ),
        out_type=jax.ShapeDtypeStruct((_HEADS, _NTYPES, _BLK, _BLK), jnp.float32),
        scratch_types=[
            pltpu.VMEM((_NUM_BUCKETS * _HEADS,), jnp.float32),
            pltpu.VMEM((_SC_PER_W,), jnp.int32),
            pltpu.VMEM((_HEADS, _SC_PER_W // _BLK, _BLK), jnp.float32),
            pltpu.SemaphoreType.DMA,
        ],
    )
    def body(tab_hbm, lut_hbm, out_hbm, tab_v, lut_v, stage_v, sem):
        wid = lax.axis_index("s") * 2 + lax.axis_index("c")
        base = pl.multiple_of(wid * _SC_PER_W, _SC_PER_W)
        nrows = _SC_PER_W // _BLK
        t = wid // (_BLK // nrows)
        r0 = pl.multiple_of((wid % (_BLK // nrows)) * nrows, nrows)
        pltpu.sync_copy(tab_hbm, tab_v)
        pltpu.sync_copy(lut_hbm.at[pl.ds(base, _SC_PER_W)], lut_v)

        def one(k, carry):
            o = pl.multiple_of(k * _SC_L, _SC_L)
            row = k // (_BLK // _SC_L)
            col = pl.multiple_of((k % (_BLK // _SC_L)) * _SC_L, _SC_L)
            idx = lut_v[pl.ds(o, _SC_L)]
            for h in range(_HEADS):
                stage_v[h, row, pl.ds(col, _SC_L)] = plsc.load_gather(
                    tab_v, [idx + h * _NUM_BUCKETS]
                )
            return carry

        lax.fori_loop(0, _SC_PER_W // _SC_L, one, 0)
        copies = [
            pltpu.async_copy(
                stage_v.at[h], out_hbm.at[h, t, pl.ds(r0, nrows)], sem
            )
            for h in range(_HEADS)
        ]
        for cp in copies:
            cp.wait()

    return body(tableT_flat, lut_flat)


_ROWS = 1024
_RSUB = _ROWS // _BLK


def _tc_add_body(qk_ref, bias_ref, out_ref):
    ib = pl.program_id(1)
    for si in range(_RSUB):
        i = ib * _RSUB + si
        rs = pl.ds(si * _BLK, _BLK)
        for j in range(_NBLK):
            d = i - j
            t = jnp.where(d == 0, 0, jnp.where(d == 1, 1, jnp.where(d < 0, 2, 3)))
            sl = pl.ds(j * _BLK, _BLK)
            out_ref[0, 0, rs, sl] = qk_ref[0, 0, rs, sl] + bias_ref[0, t]


def _tc_add(qk, bias_blocks):
    return pl.pallas_call(
        _tc_add_body,
        grid=(_HEADS, _SEQ // _ROWS),
        in_specs=[
            pl.BlockSpec((1, 1, _ROWS, _SEQ), lambda h, i: (0, h, i, 0)),
            pl.BlockSpec((1, _NTYPES, _BLK, _BLK), lambda h, i: (h, 0, 0, 0)),
        ],
        out_specs=pl.BlockSpec((1, 1, _ROWS, _SEQ), lambda h, i: (0, h, i, 0)),
        out_shape=jax.ShapeDtypeStruct(qk.shape, qk.dtype),
    )(qk, bias_blocks)


_LUT = _bucket_lut().reshape(_SC_B)


def kernel(qk_dots, table):
    tableT = jnp.reshape(jnp.transpose(table * _SCALE), (_HEADS * _NUM_BUCKETS,))
    bias = _sc_gather_bias(tableT, jnp.asarray(_LUT))
    return _tc_add(qk_dots, bias)


# speedup vs baseline: 1.5056x; 1.1243x over previous
import functools
import math

import jax
import jax.numpy as jnp
import numpy as np
from jax import lax
from jax.experimental import pallas as pl
from jax.experimental.pallas import tpu as pltpu
from jax.experimental.pallas import tpu_sc as plsc

_SCALE = 0.125
_NUM_BUCKETS = 32
_MAX_DISTANCE = 128
_HEADS = 12
_SEQ = 2048
_BLK = 256
_NBLK = _SEQ // _BLK
_NTYPES = 4
_HPAD = 16

_SC_B = _NTYPES * _BLK * _BLK
_SC_WORKERS = 32
_SC_PER_W = _SC_B // _SC_WORKERS
_SC_L = 16


def _bucket_lut() -> np.ndarray:
    bi = np.arange(_BLK, dtype=np.int64)[:, None]
    bj = np.arange(_BLK, dtype=np.int64)[None, :]
    max_exact = _NUM_BUCKETS // 2
    luts = []
    for off in (0, _BLK):
        n = np.maximum(bi - bj + off, 0)
        nf = np.maximum(n, 1).astype(np.float32)
        val = max_exact + (
            np.log(nf / np.float32(max_exact))
            / np.float32(math.log(_MAX_DISTANCE / max_exact))
            * np.float32(_NUM_BUCKETS - max_exact)
        ).astype(np.int32)
        val = np.minimum(val, _NUM_BUCKETS - 1)
        luts.append(np.where(n < max_exact, n, val).astype(np.int32))
    luts.append(np.zeros((_BLK, _BLK), np.int32))
    luts.append(np.full((_BLK, _BLK), _NUM_BUCKETS - 1, np.int32))
    return np.stack(luts)


def _sc_gather_bias(tableT_flat, lut_flat):
    mesh = plsc.VectorSubcoreMesh(core_axis_name="c", subcore_axis_name="s")

    @functools.partial(
        pl.kernel,
        mesh=mesh,
        compiler_params=pltpu.CompilerParams(
            use_tc_tiling_on_sc=False, needs_layout_passes=False
        ),
        out_type=jax.ShapeDtypeStruct((_HEADS, _NTYPES, _BLK, _BLK), jnp.float32),
        scratch_types=[
            pltpu.VMEM((_NUM_BUCKETS * _HEADS,), jnp.float32),
            pltpu.VMEM((_SC_PER_W,), jnp.int32),
            pltpu.VMEM((_HEADS, _SC_PER_W // _BLK, _BLK), jnp.float32),
            pltpu.SemaphoreType.DMA,
        ],
    )
    def body(tab_hbm, lut_hbm, out_hbm, tab_v, lut_v, stage_v, sem):
        wid = lax.axis_index("s") * 2 + lax.axis_index("c")
        base = pl.multiple_of(wid * _SC_PER_W, _SC_PER_W)
        nrows = _SC_PER_W // _BLK
        t = wid // (_BLK // nrows)
        r0 = pl.multiple_of((wid % (_BLK // nrows)) * nrows, nrows)
        pltpu.sync_copy(tab_hbm, tab_v)
        pltpu.sync_copy(lut_hbm.at[pl.ds(base, _SC_PER_W)], lut_v)

        @plsc.parallel_loop(0, _SC_PER_W // _SC_L, unroll=4)
        def one(k):
            o = pl.multiple_of(k * _SC_L, _SC_L)
            row = k // (_BLK // _SC_L)
            col = pl.multiple_of((k % (_BLK // _SC_L)) * _SC_L, _SC_L)
            idx = lut_v[pl.ds(o, _SC_L)]
            for h in range(_HEADS):
                stage_v[h, row, pl.ds(col, _SC_L)] = plsc.load_gather(
                    tab_v, [idx + h * _NUM_BUCKETS]
                )
        copies = [
            pltpu.async_copy(
                stage_v.at[h], out_hbm.at[h, t, pl.ds(r0, nrows)], sem
            )
            for h in range(_HEADS)
        ]
        for cp in copies:
            cp.wait()

    return body(tableT_flat, lut_flat)


_ROWS = 1024
_RSUB = _ROWS // _BLK


def _tc_add_body(qk_ref, bias_ref, out_ref):
    ib = pl.program_id(1)
    for si in range(_RSUB):
        i = ib * _RSUB + si
        rs = pl.ds(si * _BLK, _BLK)
        for j in range(_NBLK):
            d = i - j
            t = jnp.where(d == 0, 0, jnp.where(d == 1, 1, jnp.where(d < 0, 2, 3)))
            sl = pl.ds(j * _BLK, _BLK)
            out_ref[0, 0, rs, sl] = qk_ref[0, 0, rs, sl] + bias_ref[0, t]


def _tc_add(qk, bias_blocks):
    return pl.pallas_call(
        _tc_add_body,
        grid=(_HEADS, _SEQ // _ROWS),
        in_specs=[
            pl.BlockSpec((1, 1, _ROWS, _SEQ), lambda h, i: (0, h, i, 0)),
            pl.BlockSpec((1, _NTYPES, _BLK, _BLK), lambda h, i: (h, 0, 0, 0)),
        ],
        out_specs=pl.BlockSpec((1, 1, _ROWS, _SEQ), lambda h, i: (0, h, i, 0)),
        out_shape=jax.ShapeDtypeStruct(qk.shape, qk.dtype),
    )(qk, bias_blocks)


_LUT = _bucket_lut().reshape(_SC_B)


def kernel(qk_dots, table):
    tableT = jnp.reshape(jnp.transpose(table * _SCALE), (_HEADS * _NUM_BUCKETS,))
    bias = _sc_gather_bias(tableT, jnp.asarray(_LUT))
    return _tc_add(qk_dots, bias)
